# baseline (speedup 1.0000x reference)
import jax
import jax.numpy as jnp
from jax import lax
from jax.experimental import pallas as pl
from jax.experimental.pallas import tpu as pltpu
from jax.experimental.pallas import tpu_sc as plsc

NC, NS, L = 2, 16, 16
NW = NC * NS
B = 16384
RPW = B // NW
CHUNKS = RPW // L
GR = 64
OR = B // GR
ORW = RPW // GR

_SQRT_MAGIC = 0x1FBD1DF5


def _sqrt16(q):
    qi = lax.bitcast_convert_type(q, jnp.int32)
    y = lax.bitcast_convert_type(
        _SQRT_MAGIC + lax.shift_right_arithmetic(qi, 1), jnp.float32)
    y = 0.5 * (y + q / y)
    y = 0.5 * (y + q / y)
    return y


def _sc_body(idx_hbm, g1_hbm, g0_hbm, ew_hbm, out_hbm,
             idx_v, g1_v, g0_v, ew_v, out_v, sem):
    wid = lax.axis_index("s") * NC + lax.axis_index("c")
    rbase = wid * ORW
    pend = [
        pltpu.async_copy(idx_hbm.at[pl.ds(rbase, ORW), :], idx_v, sem),
        pltpu.async_copy(g1_hbm.at[pl.ds(rbase, ORW), :], g1_v, sem),
        pltpu.async_copy(g0_hbm.at[pl.ds(rbase, ORW), :], g0_v, sem),
        pltpu.async_copy(ew_hbm, ew_v, sem),
    ]
    for h in pend:
        h.wait()

    iota = lax.iota(jnp.int32, L)
    wm0 = ew_v[pl.ds(256, L)]
    wm1 = ew_v[pl.ds(256 + L, L)]

    def chunk(i, carry):
        row = i * L + iota
        rl = lax.shift_right_logical(row, 6)
        rc = lax.bitwise_and(row, 63)
        rc2 = rc * 2
        w1a = plsc.load_gather(g1_v, [rl, rc2])
        w1b = plsc.load_gather(g1_v, [rl, rc2 + 1])
        w0a = plsc.load_gather(g0_v, [rl, rc2])
        w0b = plsc.load_gather(g0_v, [rl, rc2 + 1])
        c0 = w0a * w1a
        c1 = w0a * w1b
        c2 = w0b * w1a
        c3 = w0b * w1b
        rc5 = rc * 5
        z = []
        for j in range(5):
            o = plsc.load_gather(idx_v, [rl, rc5 + j]) * 2
            e0a = plsc.load_gather(ew_v, [o])
            e0b = plsc.load_gather(ew_v, [o + 1])
            e1a = plsc.load_gather(ew_v, [o + 62])
            e1b = plsc.load_gather(ew_v, [o + 63])
            e2a = plsc.load_gather(ew_v, [o + 124])
            e2b = plsc.load_gather(ew_v, [o + 125])
            e3a = plsc.load_gather(ew_v, [o + 186])
            e3b = plsc.load_gather(ew_v, [o + 187])
            z.append((c0 * e0a + c1 * e1a + c2 * e2a + c3 * e3a,
                      c0 * e0b + c1 * e1b + c2 * e2b + c3 * e3b))
        qa, qb = z[0]
        s = []
        for r in range(1, 5):
            dx = qa - z[r][0]
            dy = qb - z[r][1]
            s.append(jnp.exp(-10.0 * _sqrt16(wm0 * dx * dx + wm1 * dy * dy)))
        inv = 1.0 / (s[0] + s[1] + s[2] + s[3])
        rc4 = rc * 4
        for r in range(4):
            plsc.store_scatter(out_v, [rl, rc4 + r], s[r] * inv)
        return carry

    lax.fori_loop(0, CHUNKS, chunk, 0)
    pltpu.sync_copy(out_v, out_hbm.at[pl.ds(rbase, ORW), :])


_rank_sc = pl.kernel(
    _sc_body,
    out_type=jax.ShapeDtypeStruct((OR, GR * 4), jnp.float32),
    mesh=plsc.VectorSubcoreMesh(core_axis_name="c", subcore_axis_name="s"),
    compiler_params=pltpu.CompilerParams(
        needs_layout_passes=False, use_tc_tiling_on_sc=True),
    scratch_types=[
        pltpu.VMEM((ORW, GR * 5), jnp.int32),
        pltpu.VMEM((ORW, GR * 2), jnp.float32),
        pltpu.VMEM((ORW, GR * 2), jnp.float32),
        pltpu.VMEM((288,), jnp.float32),
        pltpu.VMEM((ORW, GR * 4), jnp.float32),
        pltpu.SemaphoreType.DMA,
    ],
)


def kernel(given4rank1_stimulus_set, percept_gate_weights_1,
           percept_gate_weights_0, E0, E1, E2, E3, w_mink):
    idx = given4rank1_stimulus_set.astype(jnp.int32).reshape(OR, GR * 5)
    g1 = percept_gate_weights_1.reshape(OR, GR * 2)
    g0 = percept_gate_weights_0.reshape(OR, GR * 2)
    ew = jnp.concatenate([
        E0.reshape(-1), E1.reshape(-1), E2.reshape(-1), E3.reshape(-1),
        jnp.zeros((8,), jnp.float32),
        jnp.broadcast_to(w_mink[:, None], (2, 16)).reshape(-1),
    ])
    out = _rank_sc(idx, g1, g0, ew)
    return out.reshape(B, 4)


# speedup vs baseline: 1.2832x; 1.2832x over previous
import jax
import jax.numpy as jnp
from jax import lax
from jax.experimental import pallas as pl
from jax.experimental.pallas import tpu as pltpu
from jax.experimental.pallas import tpu_sc as plsc

NC, NS, L = 2, 16, 16
NW = NC * NS
B = 16384
RPW = B // NW
TB = 128
NTB = RPW // TB

_SQRT_MAGIC = 0x1FBD1DF5


def _sqrt16(q):
    qi = lax.bitcast_convert_type(q, jnp.int32)
    y = lax.bitcast_convert_type(
        _SQRT_MAGIC + lax.shift_right_arithmetic(qi, 1), jnp.float32)
    y = 0.5 * (y + q / y)
    y = 0.5 * (y + q / y)
    return y


def _sc_body(idx_hbm, g1_hbm, g0_hbm, ew_hbm, out_hbm,
             idx_v0, idx_v1, g1_v0, g1_v1, g0_v0, g0_v1, ew_v, out_v,
             sin0, sin1, sout):
    idx_b = [idx_v0, idx_v1]
    g1_b = [g1_v0, g1_v1]
    g0_b = [g0_v0, g0_v1]
    sin = [sin0, sin1]
    wid = lax.axis_index("s") * NC + lax.axis_index("c")
    base = wid * RPW
    pltpu.sync_copy(ew_hbm, ew_v)

    iota = lax.iota(jnp.int32, L)
    col = [jnp.full((L,), j, jnp.int32) for j in range(5)]
    zeros, ones = col[0], col[1]
    wm0 = ew_v[pl.ds(256, L)]
    wm1 = ew_v[pl.ds(256 + L, L)]

    def make_chunk(idx_v, g1_v, g0_v):
      def chunk(i, carry):
        row = i * L + iota
        w1a = plsc.load_gather(g1_v, [row, zeros])
        w1b = plsc.load_gather(g1_v, [row, ones])
        w0a = plsc.load_gather(g0_v, [row, zeros])
        w0b = plsc.load_gather(g0_v, [row, ones])
        c0 = w0a * w1a
        c1 = w0a * w1b
        c2 = w0b * w1a
        c3 = w0b * w1b
        z = []
        for j in range(5):
            o = plsc.load_gather(idx_v, [row, col[j]]) * 2
            e0a = plsc.load_gather(ew_v, [o])
            e0b = plsc.load_gather(ew_v, [o + 1])
            e1a = plsc.load_gather(ew_v, [o + 62])
            e1b = plsc.load_gather(ew_v, [o + 63])
            e2a = plsc.load_gather(ew_v, [o + 124])
            e2b = plsc.load_gather(ew_v, [o + 125])
            e3a = plsc.load_gather(ew_v, [o + 186])
            e3b = plsc.load_gather(ew_v, [o + 187])
            z.append((c0 * e0a + c1 * e1a + c2 * e2a + c3 * e3a,
                      c0 * e0b + c1 * e1b + c2 * e2b + c3 * e3b))
        qa, qb = z[0]
        s = []
        for r in range(1, 5):
            dx = qa - z[r][0]
            dy = qb - z[r][1]
            s.append(jnp.exp(-10.0 * _sqrt16(wm0 * dx * dx + wm1 * dy * dy)))
        inv = 1.0 / (s[0] + s[1] + s[2] + s[3])
        for r in range(4):
            plsc.store_scatter(out_v, [row, col[r]], s[r] * inv)
        return carry
      return chunk

    def fire_in(t, slot):
        tb = base + t * TB
        return [
            pltpu.async_copy(idx_hbm.at[pl.ds(tb, TB), :], idx_b[slot],
                             sin[slot]),
            pltpu.async_copy(g1_hbm.at[pl.ds(tb, TB), :], g1_b[slot],
                             sin[slot]),
            pltpu.async_copy(g0_hbm.at[pl.ds(tb, TB), :], g0_b[slot],
                             sin[slot]),
        ]

    in_pend = [None, None]
    out_pend = None
    in_pend[0] = fire_in(0, 0)
    for t in range(NTB):
        slot = t % 2
        if t + 1 < NTB:
            in_pend[1 - slot] = fire_in(t + 1, 1 - slot)
        for h in in_pend[slot]:
            h.wait()
        if out_pend is not None:
            out_pend.wait()
        lax.fori_loop(0, TB // L,
                      make_chunk(idx_b[slot], g1_b[slot], g0_b[slot]), 0)
        out_pend = pltpu.async_copy(
            out_v, out_hbm.at[pl.ds(base + t * TB, TB), :], sout)
    out_pend.wait()


_rank_sc = pl.kernel(
    _sc_body,
    out_type=jax.ShapeDtypeStruct((B, 4), jnp.float32),
    mesh=plsc.VectorSubcoreMesh(core_axis_name="c", subcore_axis_name="s"),
    compiler_params=pltpu.CompilerParams(
        needs_layout_passes=False, use_tc_tiling_on_sc=True),
    scratch_types=[
        pltpu.VMEM((TB, 5), jnp.int32),
        pltpu.VMEM((TB, 5), jnp.int32),
        pltpu.VMEM((TB, 2), jnp.float32),
        pltpu.VMEM((TB, 2), jnp.float32),
        pltpu.VMEM((TB, 2), jnp.float32),
        pltpu.VMEM((TB, 2), jnp.float32),
        pltpu.VMEM((288,), jnp.float32),
        pltpu.VMEM((TB, 4), jnp.float32),
        pltpu.SemaphoreType.DMA,
        pltpu.SemaphoreType.DMA,
        pltpu.SemaphoreType.DMA,
    ],
)


def kernel(given4rank1_stimulus_set, percept_gate_weights_1,
           percept_gate_weights_0, E0, E1, E2, E3, w_mink):
    ew = jnp.concatenate([
        E0.reshape(-1), E1.reshape(-1), E2.reshape(-1), E3.reshape(-1),
        jnp.zeros((8,), jnp.float32),
        jnp.broadcast_to(w_mink[:, None], (2, 16)).reshape(-1),
    ])
    return _rank_sc(given4rank1_stimulus_set.astype(jnp.int32),
                    percept_gate_weights_1, percept_gate_weights_0, ew)
